# Initial kernel scaffold; baseline (speedup 1.0000x reference)
import jax
import jax.numpy as jnp
from jax.experimental import pallas as pl


def kernel(x, edge_index, edge_attr, batch, params):
    raise NotImplementedError("write your pallas kernel here")


# speedup vs baseline: 2.2388x; 2.2388x over previous
import functools

import jax
import jax.numpy as jnp
from jax import lax
from jax.experimental import pallas as pl
from jax.experimental.pallas import tpu as pltpu
from jax.experimental.pallas import tpu_sc as plsc

N = 50000
E = 800000
EMB = 64
G = 128
L = 5

N_PAD = 50176
E_PAD = 819200
JUNK = N_PAD - 1
NB = N_PAD // 512
EB = E_PAD // 1024
ROWS_PT = N_PAD // 16
CHUNKS_ALL = E_PAD // (16 * 128)
CHUNKS_HALF = E_PAD // (32 * 128)

_MESH = dict(mesh=plsc.VectorSubcoreMesh(core_axis_name="c", subcore_axis_name="s"),
             compiler_params=pltpu.CompilerParams(use_tc_tiling_on_sc=False))


@functools.partial(
    pl.kernel,
    out_type=[jax.ShapeDtypeStruct((N_PAD, 16), jnp.float32),
              jax.ShapeDtypeStruct((N_PAD, 16), jnp.float32)],
    scratch_types=[pltpu.VMEM((CHUNKS_HALF, 128), jnp.int32),
                   pltpu.VMEM((128, 16), jnp.float32),
                   pltpu.VMEM_SHARED((N_PAD, 16), jnp.float32)],
    **_MESH,
)
def _sc_deg(row2, z16, ones_in, degA, degB, idxs, onesv, table):
    c = lax.axis_index("c")
    s = lax.axis_index("s")
    nb = s * ROWS_PT
    pltpu.sync_copy(z16.at[pl.ds(nb, ROWS_PT), :], table.at[pl.ds(nb, ROWS_PT), :])
    pltpu.sync_copy(ones_in, onesv)
    cbase = (c * 16 + s) * CHUNKS_HALF
    pltpu.sync_copy(row2.at[pl.ds(cbase, CHUNKS_HALF), :], idxs)
    plsc.subcore_barrier()

    def chunk(j, carry):
        pltpu.sync_copy(onesv, table.at[idxs.at[j]], add=True)
        return carry

    lax.fori_loop(0, CHUNKS_HALF, chunk, 0)
    plsc.subcore_barrier()

    @pl.when(c == 0)
    def _():
        pltpu.sync_copy(table.at[pl.ds(nb, ROWS_PT), :], degA.at[pl.ds(nb, ROWS_PT), :])

    @pl.when(c == 1)
    def _():
        pltpu.sync_copy(table.at[pl.ds(nb, ROWS_PT), :], degB.at[pl.ds(nb, ROWS_PT), :])


@functools.partial(
    pl.kernel,
    out_type=jax.ShapeDtypeStruct((E_PAD,), jnp.float32),
    scratch_types=[pltpu.VMEM((CHUNKS_HALF, 128), jnp.int32),
                   pltpu.VMEM((CHUNKS_HALF * 128,), jnp.float32),
                   pltpu.SemaphoreType.DMA],
    **_MESH,
)
def _sc_disrow(row2, disf, disrow, idxs, outv, sem):
    c = lax.axis_index("c")
    s = lax.axis_index("s")
    wid = c * 16 + s
    pltpu.sync_copy(row2.at[pl.ds(wid * CHUNKS_HALF, CHUNKS_HALF), :], idxs)

    def chunk(j, carry):
        pltpu.async_copy(disf.at[idxs.at[j]], outv.at[pl.ds(j * 128, 128)], sem).wait()
        return carry

    lax.fori_loop(0, CHUNKS_HALF, chunk, 0)
    pltpu.sync_copy(outv, disrow.at[pl.ds(wid * CHUNKS_HALF * 128, CHUNKS_HALF * 128)])


@functools.partial(
    pl.kernel,
    out_type=[jax.ShapeDtypeStruct((N_PAD, 16), jnp.float32) for _ in range(4)],
    scratch_types=[pltpu.VMEM((CHUNKS_ALL // 2, 128), jnp.int32),
                   pltpu.VMEM((CHUNKS_ALL // 2, 128), jnp.int32),
                   pltpu.VMEM((128, 16), jnp.float32),
                   pltpu.VMEM((128, 16), jnp.float32),
                   pltpu.VMEM((128, 16), jnp.float32),
                   pltpu.VMEM_SHARED((N_PAD, 16), jnp.float32),
                   pltpu.SemaphoreType.DMA],
    **_MESH,
)
def _sc_msgpass(hs0, hs1, hs2, hs3, eas0, eas1, eas2, eas3, row2, col2, z16,
                agg0, agg1, agg2, agg3,
                rowi, coli, gbuf, eabuf, msgb, acc, sem):
    c = lax.axis_index("c")
    s = lax.axis_index("s")
    nb = s * ROWS_PT
    qc = CHUNKS_ALL // 2
    hs_q = [hs0, hs1, hs2, hs3]
    eas_q = [eas0, eas1, eas2, eas3]
    agg_q = [agg0, agg1, agg2, agg3]

    def run(cval):
        for q in range(2):
            fq = 2 * cval + q
            hs_ref, eas_ref, agg_ref = hs_q[fq], eas_q[fq], agg_q[fq]
            pltpu.sync_copy(z16.at[pl.ds(nb, ROWS_PT), :], acc.at[pl.ds(nb, ROWS_PT), :])
            plsc.subcore_barrier()
            for p in range(2):
                cbase = s * CHUNKS_ALL + p * qc
                pltpu.sync_copy(row2.at[pl.ds(cbase, qc), :], rowi)
                pltpu.sync_copy(col2.at[pl.ds(cbase, qc), :], coli)

                def chunk(j, carry):
                    pltpu.async_copy(hs_ref.at[rowi.at[j]], gbuf, sem).wait()
                    ebase = (cbase + j) * 128
                    pltpu.sync_copy(eas_ref.at[pl.ds(ebase, 128), :], eabuf)

                    def rowfn(r, rc):
                        msgb[r, pl.ds(0, 16)] = jnp.maximum(
                            gbuf[r, pl.ds(0, 16)] + eabuf[r, pl.ds(0, 16)], 0.0)
                        return rc

                    lax.fori_loop(0, 128, rowfn, 0)
                    pltpu.sync_copy(msgb, acc.at[coli.at[j]], add=True)
                    return carry

                lax.fori_loop(0, qc, chunk, 0)
            plsc.subcore_barrier()
            pltpu.sync_copy(acc.at[pl.ds(nb, ROWS_PT), :], agg_ref.at[pl.ds(nb, ROWS_PT), :])
            plsc.subcore_barrier()

    @pl.when(c == 0)
    def _():
        run(0)

    @pl.when(c == 1)
    def _():
        run(1)


def _k0_body(x, w, b, out):
    out[...] = jnp.dot(x[...], w[...].T, preferred_element_type=jnp.float32) + b[...]


def _k1_body(dA, dB, dis, inv):
    d = dA[:, 0:1] + dB[:, 0:1] + 1.0
    dis[...] = lax.rsqrt(d)
    inv[...] = 1.0 / d


def _k2_body(ea, dr, w, b, o0, o1, o2, o3):
    z = jnp.dot(ea[...], w[...].T, preferred_element_type=jnp.float32) + b[...]
    z = z * dr[...]
    for q, o in enumerate((o0, o1, o2, o3)):
        o[...] = z[:, q * 16:(q + 1) * 16]


def _k3_body(h, w, b, dis, hlin, hs0, hs1, hs2, hs3):
    z = jnp.dot(h[...], w[...].T, preferred_element_type=jnp.float32) + b[...]
    hlin[...] = z
    hs = z * dis[...]
    for q, o in enumerate((hs0, hs1, hs2, hs3)):
        o[...] = hs[:, q * 16:(q + 1) * 16]


def _k4a_body(hlin, agg0, agg1, agg2, agg3, hin, dis, inv, root, hnew, ps, pq):
    j = pl.program_id(0)
    agg = jnp.concatenate([agg0[...], agg1[...], agg2[...], agg3[...]],
                          axis=1) * dis[...]
    selfterm = jnp.maximum(hlin[...] + root[...], 0.0) * inv[...]
    t = hin[...] + agg + selfterm
    hnew[...] = t
    rows = j * 512 + lax.broadcasted_iota(jnp.int32, (512, 1), 0)
    tm = jnp.where(rows < N, t, 0.0)
    ps[...] = jnp.sum(tm, axis=0)[None, None, :]
    pq[...] = jnp.sum(tm * tm, axis=0)[None, None, :]


def _k4c_body(hnew, ps, pq, g, b, out, *, do_relu):
    mu = jnp.sum(ps[...], axis=(0, 1)) / N
    ex2 = jnp.sum(pq[...], axis=(0, 1)) / N
    var = ex2 - mu * mu
    y = (hnew[...] - mu[None, :]) * lax.rsqrt(var[None, :] + 1e-5)
    y = y * g[...] + b[...]
    if do_relu:
        y = jnp.maximum(y, 0.0)
    out[...] = y


def _k5_body(h, bt, sums, cnt):
    j = pl.program_id(0)

    @pl.when(j == 0)
    def _():
        sums[...] = jnp.zeros_like(sums)
        cnt[...] = jnp.zeros_like(cnt)

    oh = (bt[...] == lax.broadcasted_iota(jnp.int32, (1, G), 1)).astype(jnp.float32)
    dn = (((0,), (0,)), ((), ()))
    sums[...] += lax.dot_general(oh, h[...], dn, preferred_element_type=jnp.float32)
    cnt[...] += lax.dot_general(oh, jnp.ones((512, EMB), jnp.float32), dn,
                                preferred_element_type=jnp.float32)


def _k6_body(sums, cnt, w1, b1, w2, b2, w3, b3, out):
    hg = sums[...] * (1.0 / jnp.maximum(cnt[...], 1.0))
    dn = (((1,), (1,)), ((), ()))
    z1 = jnp.maximum(lax.dot_general(hg, w1[...], dn, preferred_element_type=jnp.float32) + b1[...], 0.0)
    z2 = jnp.maximum(lax.dot_general(z1, w2[...], dn, preferred_element_type=jnp.float32) + b2[...], 0.0)
    out[...] = lax.dot_general(z2, w3[...], dn, preferred_element_type=jnp.float32) + b3[0, 0]


def _k6_specs():
    H = EMB // 2
    return dict(
        in_specs=[_full((G, EMB)), _full((G, EMB)), _full((H, EMB)), _full((1, H)),
                  _full((H, H)), _full((1, H)), _full((G, H)), _full((1, 1))],
        out_specs=_full((G, G)),
        out_shape=jax.ShapeDtypeStruct((G, G), _f32),
    )


def _full(shape):
    return pl.BlockSpec(shape, lambda *a: tuple(0 for _ in shape))


def _rows(shape):
    nd = len(shape)
    return pl.BlockSpec(shape, lambda j: (j,) + tuple(0 for _ in range(nd - 1)))


_f32 = jnp.float32


def _tc_embed(x, w, b):
    return pl.pallas_call(
        _k0_body, grid=(NB,),
        in_specs=[_rows((512, 40)), _full((EMB, 40)), _full((1, EMB))],
        out_specs=_rows((512, EMB)),
        out_shape=jax.ShapeDtypeStruct((N_PAD, EMB), _f32),
    )(x, w, b)


def _tc_dis(dA, dB):
    return pl.pallas_call(
        _k1_body, grid=(NB,),
        in_specs=[_rows((512, 16)), _rows((512, 16))],
        out_specs=[_rows((512, 1)), _rows((512, 1))],
        out_shape=[jax.ShapeDtypeStruct((N_PAD, 1), _f32),
                   jax.ShapeDtypeStruct((N_PAD, 1), _f32)],
    )(dA, dB)


def _tc_eas(ea, dr, w, b):
    return pl.pallas_call(
        _k2_body, grid=(EB,),
        in_specs=[_rows((1024, 10)), _rows((1024, 1)), _full((EMB, 10)), _full((1, EMB))],
        out_specs=[_rows((1024, 16)) for _ in range(4)],
        out_shape=[jax.ShapeDtypeStruct((E_PAD, 16), _f32) for _ in range(4)],
    )(ea, dr, w, b)


def _tc_lin(h, w, b, dis):
    return pl.pallas_call(
        _k3_body, grid=(NB,),
        in_specs=[_rows((512, EMB)), _full((EMB, EMB)), _full((1, EMB)), _rows((512, 1))],
        out_specs=[_rows((512, EMB))] + [_rows((512, 16)) for _ in range(4)],
        out_shape=[jax.ShapeDtypeStruct((N_PAD, EMB), _f32)]
                  + [jax.ShapeDtypeStruct((N_PAD, 16), _f32) for _ in range(4)],
    )(h, w, b, dis)


def _tc_combine(hlin, aggq, hin, dis, inv, root):
    return pl.pallas_call(
        _k4a_body, grid=(NB,),
        in_specs=[_rows((512, EMB))] + [_rows((512, 16)) for _ in range(4)]
                 + [_rows((512, EMB)), _rows((512, 1)), _rows((512, 1)), _full((1, EMB))],
        out_specs=[_rows((512, EMB)),
                   pl.BlockSpec((1, 1, EMB), lambda j: (j, 0, 0)),
                   pl.BlockSpec((1, 1, EMB), lambda j: (j, 0, 0))],
        out_shape=[jax.ShapeDtypeStruct((N_PAD, EMB), _f32),
                   jax.ShapeDtypeStruct((NB, 1, EMB), _f32),
                   jax.ShapeDtypeStruct((NB, 1, EMB), _f32)],
    )(hlin, *aggq, hin, dis, inv, root)


def _tc_bn(hnew, ps, pq, g, b, do_relu):
    return pl.pallas_call(
        functools.partial(_k4c_body, do_relu=do_relu), grid=(NB,),
        in_specs=[_rows((512, EMB)), _full((NB, 1, EMB)), _full((NB, 1, EMB)),
                  _full((1, EMB)), _full((1, EMB))],
        out_specs=_rows((512, EMB)),
        out_shape=jax.ShapeDtypeStruct((N_PAD, EMB), _f32),
    )(hnew, ps, pq, g, b)


def _tc_pool(h, bt):
    return pl.pallas_call(
        _k5_body, grid=(NB,),
        in_specs=[_rows((512, EMB)), _rows((512, 1))],
        out_specs=[_full((G, EMB)), _full((G, EMB))],
        out_shape=[jax.ShapeDtypeStruct((G, EMB), _f32),
                   jax.ShapeDtypeStruct((G, EMB), _f32)],
    )(h, bt)


def _tc_mlp(sums, cnt, w1, b1, w2, b2, w3, b3):
    return pl.pallas_call(_k6_body, grid=(1,), **_k6_specs())(
        sums, cnt, w1, b1, w2, b2, w3, b3)


@jax.jit
def _stage_pre(x_p, row2, ea_p, params):
    z16 = jnp.zeros((N_PAD, 16), _f32)
    ones16 = jnp.ones((128, 16), _f32)
    degA, degB = _sc_deg(row2, z16, ones16)
    dis2, inv2 = _tc_dis(degA, degB)
    disrow = _sc_disrow(row2, dis2.reshape(N_PAD))
    easq = _tc_eas(ea_p, disrow.reshape(E_PAD, 1),
                   params['edge_emb_W'], params['edge_emb_b'].reshape(1, EMB))
    h = _tc_embed(x_p, params['x_emb_W'], params['x_emb_b'].reshape(1, EMB))
    return h, dis2, inv2, easq


@jax.jit
def _stage_layers(h, dis2, inv2, easq, row2, col2, bt_p, params):
    z16 = jnp.zeros((N_PAD, 16), _f32)
    for l in range(L):
        lp = params['layers'][l]
        hlin, *hsq = _tc_lin(h, lp['lin_W'], lp['lin_b'].reshape(1, EMB), dis2)
        aggq = _sc_msgpass(*hsq, *easq, row2, col2, z16)
        hnew, ps, pq = _tc_combine(hlin, aggq, h, dis2, inv2, lp['root'])
        h = _tc_bn(hnew, ps, pq, lp['bn_g'].reshape(1, EMB),
                   lp['bn_b'].reshape(1, EMB), do_relu=(l < L - 1))

    sums, cnt = _tc_pool(h, bt_p)
    (w1, b1), (w2, b2), (w3, b3) = params['pred']
    H = EMB // 2
    w3p = jnp.pad(w3, ((0, G - 1), (0, 0)))
    full = _tc_mlp(sums, cnt, w1, b1.reshape(1, H), w2, b2.reshape(1, H),
                   w3p, b3.reshape(1, 1))
    return full[:, :1]


def kernel(x, edge_index, edge_attr, batch, params):
    x_p = jnp.pad(x, ((0, N_PAD - N), (0, 0)))
    row = jnp.pad(edge_index[0], (0, E_PAD - E), constant_values=JUNK)
    col = jnp.pad(edge_index[1], (0, E_PAD - E), constant_values=JUNK)
    ea_p = jnp.pad(edge_attr, ((0, E_PAD - E), (0, 0)))
    bt_p = jnp.pad(batch, (0, N_PAD - N), constant_values=G).reshape(N_PAD, 1)
    row2 = row.reshape(E_PAD // 128, 128)
    col2 = col.reshape(E_PAD // 128, 128)
    h, dis2, inv2, easq = _stage_pre(x_p, row2, ea_p, params)
    return _stage_layers(h, dis2, inv2, easq, row2, col2, bt_p, params)


# speedup vs baseline: 3.9620x; 1.7697x over previous
import functools

import jax
import jax.numpy as jnp
from jax import lax
from jax.experimental import pallas as pl
from jax.experimental.pallas import tpu as pltpu
from jax.experimental.pallas import tpu_sc as plsc

N = 50000
E = 800000
EMB = 64
G = 128
L = 5

N_PAD = 50176
E_PAD = 819200
JUNK = N_PAD - 1
NB = N_PAD // 512
EB = E_PAD // 1024
ROWS_PT = N_PAD // 16
CHUNKS_ALL = E_PAD // (16 * 128)
CHUNKS_HALF = E_PAD // (32 * 128)

_MESH = dict(mesh=plsc.VectorSubcoreMesh(core_axis_name="c", subcore_axis_name="s"),
             compiler_params=pltpu.CompilerParams(use_tc_tiling_on_sc=False))
_DEPTH = 4


@functools.partial(
    pl.kernel,
    out_type=[jax.ShapeDtypeStruct((N_PAD,), jnp.float32),
              jax.ShapeDtypeStruct((N_PAD,), jnp.float32)],
    scratch_types=[pltpu.VMEM((CHUNKS_HALF, 128), jnp.int32),
                   pltpu.VMEM((128,), jnp.float32),
                   pltpu.VMEM_SHARED((N_PAD,), jnp.float32),
                   pltpu.SemaphoreType.DMA],
    **_MESH,
)
def _sc_deg(row2, z1, ones1, degA, degB, idxs, onesv, table, sem):
    c = lax.axis_index("c")
    s = lax.axis_index("s")
    nb = s * ROWS_PT
    pltpu.sync_copy(z1.at[pl.ds(nb, ROWS_PT)], table.at[pl.ds(nb, ROWS_PT)])
    pltpu.sync_copy(ones1, onesv)
    pltpu.sync_copy(row2.at[pl.ds((c * 16 + s) * CHUNKS_HALF, CHUNKS_HALF), :], idxs)
    plsc.subcore_barrier()

    def step(t, carry):
        for u in range(8):
            pltpu.async_copy(onesv, table.at[idxs.at[t * 8 + u]], add=True, sem=sem)
        for u in range(8):
            pltpu.make_async_copy(onesv, table.at[idxs.at[0]], sem).wait()
        return carry

    lax.fori_loop(0, CHUNKS_HALF // 8, step, 0)
    plsc.subcore_barrier()

    @pl.when(c == 0)
    def _():
        pltpu.sync_copy(table.at[pl.ds(nb, ROWS_PT)], degA.at[pl.ds(nb, ROWS_PT)])

    @pl.when(c == 1)
    def _():
        pltpu.sync_copy(table.at[pl.ds(nb, ROWS_PT)], degB.at[pl.ds(nb, ROWS_PT)])


@functools.partial(
    pl.kernel,
    out_type=jax.ShapeDtypeStruct((E_PAD,), jnp.float32),
    scratch_types=[pltpu.VMEM((CHUNKS_HALF, 128), jnp.int32),
                   pltpu.VMEM((CHUNKS_HALF * 128,), jnp.float32),
                   pltpu.SemaphoreType.DMA],
    **_MESH,
)
def _sc_disrow(row2, disf, disrow, idxs, outv, sem):
    c = lax.axis_index("c")
    s = lax.axis_index("s")
    wid = c * 16 + s
    pltpu.sync_copy(row2.at[pl.ds(wid * CHUNKS_HALF, CHUNKS_HALF), :], idxs)

    def chunk(j, carry):
        pltpu.async_copy(disf.at[idxs.at[j]], outv.at[pl.ds(j * 128, 128)], sem).wait()
        return carry

    lax.fori_loop(0, CHUNKS_HALF, chunk, 0)
    pltpu.sync_copy(outv, disrow.at[pl.ds(wid * CHUNKS_HALF * 128, CHUNKS_HALF * 128)])


@functools.partial(
    pl.kernel,
    out_type=[jax.ShapeDtypeStruct((N_PAD, 16), jnp.float32) for _ in range(4)],
    scratch_types=[pltpu.VMEM((CHUNKS_ALL // 2, 128), jnp.int32),
                   pltpu.VMEM((CHUNKS_ALL // 2, 128), jnp.int32)]
                  + [pltpu.VMEM((128, 16), jnp.float32) for _ in range(3 * _DEPTH)]
                  + [pltpu.VMEM_SHARED((N_PAD, 16), jnp.float32)]
                  + [pltpu.SemaphoreType.DMA for _ in range(2 * _DEPTH)],
    **_MESH,
)
def _sc_msgpass(hs0, hs1, hs2, hs3, eas0, eas1, eas2, eas3, row2, col2, z16,
                agg0, agg1, agg2, agg3, ri, ci, *bufs):
    DEPTH = _DEPTH
    gq = bufs[0:DEPTH]
    eq = bufs[DEPTH:2 * DEPTH]
    mq = bufs[2 * DEPTH:3 * DEPTH]
    acc = bufs[3 * DEPTH]
    sgeq = bufs[3 * DEPTH + 1:4 * DEPTH + 1]
    ssq = bufs[4 * DEPTH + 1:5 * DEPTH + 1]
    c = lax.axis_index("c")
    s = lax.axis_index("s")
    nb = s * ROWS_PT
    qc = CHUNKS_ALL // 2

    def run(cval):
        for q in range(2):
            fq = 2 * cval + q
            hs_ref = (hs0, hs1, hs2, hs3)[fq]
            eas_ref = (eas0, eas1, eas2, eas3)[fq]
            agg_ref = (agg0, agg1, agg2, agg3)[fq]
            pltpu.sync_copy(z16.at[pl.ds(nb, ROWS_PT), :], acc.at[pl.ds(nb, ROWS_PT), :])
            plsc.subcore_barrier()

            def fire(j, p, ebase0):
                pltpu.async_copy(hs_ref.at[ri.at[j]], gq[p], sgeq[p])
                pltpu.async_copy(eas_ref.at[pl.ds(ebase0 + j * 128, 128), :], eq[p], sgeq[p])

            def wait_ge(p):
                pltpu.make_async_copy(hs_ref.at[ri.at[0]], gq[p], sgeq[p]).wait()
                pltpu.make_async_copy(eas_ref.at[pl.ds(0, 128), :], eq[p], sgeq[p]).wait()

            def compute_and_scatter(j, p):
                def rowfn(r, rc):
                    mq[p][r, pl.ds(0, 16)] = jnp.maximum(
                        gq[p][r, pl.ds(0, 16)] + eq[p][r, pl.ds(0, 16)], 0.0)
                    return rc
                lax.fori_loop(0, 128, rowfn, 0)
                pltpu.async_copy(mq[p], acc.at[ci.at[j]], add=True, sem=ssq[p])

            def drain_sc(p):
                pltpu.make_async_copy(mq[p], acc.at[ci.at[0]], ssq[p]).wait()

            for pc in range(2):
                cbase = s * CHUNKS_ALL + pc * qc
                ebase0 = cbase * 128
                pltpu.sync_copy(row2.at[pl.ds(cbase, qc), :], ri)
                pltpu.sync_copy(col2.at[pl.ds(cbase, qc), :], ci)
                for u in range(DEPTH - 1):
                    fire(u, u, ebase0)

                def step(t, carry):
                    for u in range(DEPTH):
                        j = t * DEPTH + u
                        pl.when(t > 0)(lambda: drain_sc(u))
                        wait_ge(u)
                        jn = j + DEPTH - 1
                        pl.when(jn < qc)(lambda: fire(jn, (u + DEPTH - 1) % DEPTH, ebase0))
                        compute_and_scatter(j, u)
                    return carry

                lax.fori_loop(0, qc // DEPTH, step, 0)
                for u in range(DEPTH):
                    drain_sc(u)
            plsc.subcore_barrier()
            pltpu.sync_copy(acc.at[pl.ds(nb, ROWS_PT), :], agg_ref.at[pl.ds(nb, ROWS_PT), :])
            plsc.subcore_barrier()

    @pl.when(c == 0)
    def _():
        run(0)

    @pl.when(c == 1)
    def _():
        run(1)


def _k0_body(x, w, b, out):
    out[...] = jnp.dot(x[...], w[...].T, preferred_element_type=jnp.float32) + b[...]


def _k1_body(dA, dB, dis, inv):
    d = dA[...] + dB[...] + 1.0
    dis[...] = lax.rsqrt(d)
    inv[...] = 1.0 / d


def _k2_body(ea, dr, w, b, o0, o1, o2, o3):
    z = jnp.dot(ea[...], w[...].T, preferred_element_type=jnp.float32) + b[...]
    z = z * dr[...]
    for q, o in enumerate((o0, o1, o2, o3)):
        o[...] = z[:, q * 16:(q + 1) * 16]


def _k3_body(h, w, b, dis, hlin, hs0, hs1, hs2, hs3):
    z = jnp.dot(h[...], w[...].T, preferred_element_type=jnp.float32) + b[...]
    hlin[...] = z
    hs = z * dis[...]
    for q, o in enumerate((hs0, hs1, hs2, hs3)):
        o[...] = hs[:, q * 16:(q + 1) * 16]


def _k4a_body(hlin, agg0, agg1, agg2, agg3, hin, dis, inv, root, hnew, ps, pq):
    j = pl.program_id(0)
    agg = jnp.concatenate([agg0[...], agg1[...], agg2[...], agg3[...]],
                          axis=1) * dis[...]
    selfterm = jnp.maximum(hlin[...] + root[...], 0.0) * inv[...]
    t = hin[...] + agg + selfterm
    hnew[...] = t
    rows = j * 512 + lax.broadcasted_iota(jnp.int32, (512, 1), 0)
    tm = jnp.where(rows < N, t, 0.0)
    ps[...] = jnp.sum(tm, axis=0)[None, None, :]
    pq[...] = jnp.sum(tm * tm, axis=0)[None, None, :]


def _k4c_body(hnew, ps, pq, g, b, m, out):
    mu = jnp.sum(ps[...], axis=(0, 1)) / N
    ex2 = jnp.sum(pq[...], axis=(0, 1)) / N
    var = ex2 - mu * mu
    y = (hnew[...] - mu[None, :]) * lax.rsqrt(var[None, :] + 1e-5)
    y = y * g[...] + b[...]
    out[...] = jnp.maximum(y, y * m[...])


def _k5_body(h, bt, sums, cnt):
    j = pl.program_id(0)

    @pl.when(j == 0)
    def _():
        sums[...] = jnp.zeros_like(sums)
        cnt[...] = jnp.zeros_like(cnt)

    oh = (bt[...] == lax.broadcasted_iota(jnp.int32, (1, G), 1)).astype(jnp.float32)
    dn = (((0,), (0,)), ((), ()))
    sums[...] += lax.dot_general(oh, h[...], dn, preferred_element_type=jnp.float32)
    cnt[...] += lax.dot_general(oh, jnp.ones((512, EMB), jnp.float32), dn,
                                preferred_element_type=jnp.float32)


def _k6_body(sums, cnt, w1, b1, w2, b2, w3, b3, out):
    hg = sums[...] * (1.0 / jnp.maximum(cnt[...], 1.0))
    dn = (((1,), (1,)), ((), ()))
    z1 = jnp.maximum(lax.dot_general(hg, w1[...], dn, preferred_element_type=jnp.float32) + b1[...], 0.0)
    z2 = jnp.maximum(lax.dot_general(z1, w2[...], dn, preferred_element_type=jnp.float32) + b2[...], 0.0)
    out[...] = lax.dot_general(z2, w3[...], dn, preferred_element_type=jnp.float32) + b3[0, 0]


def _k6_specs():
    H = EMB // 2
    return dict(
        in_specs=[_full((G, EMB)), _full((G, EMB)), _full((H, EMB)), _full((1, H)),
                  _full((H, H)), _full((1, H)), _full((G, H)), _full((1, 1))],
        out_specs=_full((G, G)),
        out_shape=jax.ShapeDtypeStruct((G, G), _f32),
    )


def _full(shape):
    return pl.BlockSpec(shape, lambda *a: tuple(0 for _ in shape))


def _rows(shape):
    nd = len(shape)
    return pl.BlockSpec(shape, lambda j: (j,) + tuple(0 for _ in range(nd - 1)))


_f32 = jnp.float32


def _tc_embed(x, w, b):
    return pl.pallas_call(
        _k0_body, grid=(NB,),
        in_specs=[_rows((512, 40)), _full((EMB, 40)), _full((1, EMB))],
        out_specs=_rows((512, EMB)),
        out_shape=jax.ShapeDtypeStruct((N_PAD, EMB), _f32),
    )(x, w, b)


def _tc_dis(dA, dB):
    return pl.pallas_call(
        _k1_body, grid=(NB,),
        in_specs=[_rows((512, 1)), _rows((512, 1))],
        out_specs=[_rows((512, 1)), _rows((512, 1))],
        out_shape=[jax.ShapeDtypeStruct((N_PAD, 1), _f32),
                   jax.ShapeDtypeStruct((N_PAD, 1), _f32)],
    )(dA, dB)


def _tc_eas(ea, dr, w, b):
    return pl.pallas_call(
        _k2_body, grid=(EB,),
        in_specs=[_rows((1024, 10)), _rows((1024, 1)), _full((EMB, 10)), _full((1, EMB))],
        out_specs=[_rows((1024, 16)) for _ in range(4)],
        out_shape=[jax.ShapeDtypeStruct((E_PAD, 16), _f32) for _ in range(4)],
    )(ea, dr, w, b)


def _tc_lin(h, w, b, dis):
    return pl.pallas_call(
        _k3_body, grid=(NB,),
        in_specs=[_rows((512, EMB)), _full((EMB, EMB)), _full((1, EMB)), _rows((512, 1))],
        out_specs=[_rows((512, EMB))] + [_rows((512, 16)) for _ in range(4)],
        out_shape=[jax.ShapeDtypeStruct((N_PAD, EMB), _f32)]
                  + [jax.ShapeDtypeStruct((N_PAD, 16), _f32) for _ in range(4)],
    )(h, w, b, dis)


def _tc_combine(hlin, aggq, hin, dis, inv, root):
    return pl.pallas_call(
        _k4a_body, grid=(NB,),
        in_specs=[_rows((512, EMB))] + [_rows((512, 16)) for _ in range(4)]
                 + [_rows((512, EMB)), _rows((512, 1)), _rows((512, 1)), _full((1, EMB))],
        out_specs=[_rows((512, EMB)),
                   pl.BlockSpec((1, 1, EMB), lambda j: (j, 0, 0)),
                   pl.BlockSpec((1, 1, EMB), lambda j: (j, 0, 0))],
        out_shape=[jax.ShapeDtypeStruct((N_PAD, EMB), _f32),
                   jax.ShapeDtypeStruct((NB, 1, EMB), _f32),
                   jax.ShapeDtypeStruct((NB, 1, EMB), _f32)],
    )(hlin, *aggq, hin, dis, inv, root)


def _tc_bn(hnew, ps, pq, g, b, m):
    return pl.pallas_call(
        _k4c_body, grid=(NB,),
        in_specs=[_rows((512, EMB)), _full((NB, 1, EMB)), _full((NB, 1, EMB)),
                  _full((1, EMB)), _full((1, EMB)), _full((1, EMB))],
        out_specs=_rows((512, EMB)),
        out_shape=jax.ShapeDtypeStruct((N_PAD, EMB), _f32),
    )(hnew, ps, pq, g, b, m)


def _tc_pool(h, bt):
    return pl.pallas_call(
        _k5_body, grid=(NB,),
        in_specs=[_rows((512, EMB)), _rows((512, 1))],
        out_specs=[_full((G, EMB)), _full((G, EMB))],
        out_shape=[jax.ShapeDtypeStruct((G, EMB), _f32),
                   jax.ShapeDtypeStruct((G, EMB), _f32)],
    )(h, bt)


def _tc_mlp(sums, cnt, w1, b1, w2, b2, w3, b3):
    return pl.pallas_call(_k6_body, grid=(1,), **_k6_specs())(
        sums, cnt, w1, b1, w2, b2, w3, b3)


@jax.jit
def _stage_pre(x_p, row2, ea_p, params):
    z1 = jnp.zeros((N_PAD,), _f32)
    ones1 = jnp.ones((128,), _f32)
    degA, degB = _sc_deg(row2, z1, ones1)
    dis2, inv2 = _tc_dis(degA.reshape(N_PAD, 1), degB.reshape(N_PAD, 1))
    disrow = _sc_disrow(row2, dis2.reshape(N_PAD))
    easq = _tc_eas(ea_p, disrow.reshape(E_PAD, 1),
                   params['edge_emb_W'], params['edge_emb_b'].reshape(1, EMB))
    h = _tc_embed(x_p, params['x_emb_W'], params['x_emb_b'].reshape(1, EMB))
    return h, dis2, inv2, easq


@jax.jit
def _stage_layers(h, dis2, inv2, easq, row2, col2, bt_p, params):
    z16 = jnp.zeros((N_PAD, 16), _f32)
    lps = params['layers']
    Ws = jnp.stack([lp['lin_W'] for lp in lps])
    bs = jnp.stack([lp['lin_b'].reshape(1, EMB) for lp in lps])
    roots = jnp.stack([lp['root'] for lp in lps])
    gs = jnp.stack([lp['bn_g'].reshape(1, EMB) for lp in lps])
    betas = jnp.stack([lp['bn_b'].reshape(1, EMB) for lp in lps])
    relu_m = jnp.concatenate([jnp.zeros((L - 1, 1, EMB), _f32),
                              jnp.ones((1, 1, EMB), _f32)])

    for l in range(L):
        idx = lambda a: a[l]
        hlin, *hsq = _tc_lin(h, idx(Ws), idx(bs), dis2)
        aggq = _sc_msgpass(*hsq, *easq, row2, col2, z16)
        hnew, ps, pq = _tc_combine(hlin, aggq, h, dis2, inv2, idx(roots))
        h = _tc_bn(hnew, ps, pq, idx(gs), idx(betas), idx(relu_m))

    sums, cnt = _tc_pool(h, bt_p)
    (w1, b1), (w2, b2), (w3, b3) = params['pred']
    H = EMB // 2
    w3p = jnp.pad(w3, ((0, G - 1), (0, 0)))
    full = _tc_mlp(sums, cnt, w1, b1.reshape(1, H), w2, b2.reshape(1, H),
                   w3p, b3.reshape(1, 1))
    return full[:, :1]


def kernel(x, edge_index, edge_attr, batch, params):
    x_p = jnp.pad(x, ((0, N_PAD - N), (0, 0)))
    row = jnp.pad(edge_index[0], (0, E_PAD - E), constant_values=JUNK)
    col = jnp.pad(edge_index[1], (0, E_PAD - E), constant_values=JUNK)
    ea_p = jnp.pad(edge_attr, ((0, E_PAD - E), (0, 0)))
    bt_p = jnp.pad(batch, (0, N_PAD - N), constant_values=G).reshape(N_PAD, 1)
    row2 = row.reshape(E_PAD // 128, 128)
    col2 = col.reshape(E_PAD // 128, 128)
    h, dis2, inv2, easq = _stage_pre(x_p, row2, ea_p, params)
    return _stage_layers(h, dis2, inv2, easq, row2, col2, bt_p, params)


# speedup vs baseline: 4.1433x; 1.0458x over previous
import functools

import jax
import jax.numpy as jnp
from jax import lax
from jax.experimental import pallas as pl
from jax.experimental.pallas import tpu as pltpu
from jax.experimental.pallas import tpu_sc as plsc

N = 50000
E = 800000
EMB = 64
G = 128
L = 5

N_PAD = 50176
E_PAD = 819200
JUNK = N_PAD - 1
NB = N_PAD // 512
EB = E_PAD // 1024
ROWS_PT = N_PAD // 16
CHUNKS_ALL = E_PAD // (16 * 128)
CHUNKS_HALF = E_PAD // (32 * 128)

_MESH = dict(mesh=plsc.VectorSubcoreMesh(core_axis_name="c", subcore_axis_name="s"),
             compiler_params=pltpu.CompilerParams(use_tc_tiling_on_sc=False))
_DEPTH = 4


@functools.partial(
    pl.kernel,
    out_type=[jax.ShapeDtypeStruct((N_PAD,), jnp.float32),
              jax.ShapeDtypeStruct((N_PAD,), jnp.float32)],
    scratch_types=[pltpu.VMEM((CHUNKS_HALF, 128), jnp.int32),
                   pltpu.VMEM((128,), jnp.float32),
                   pltpu.VMEM_SHARED((N_PAD,), jnp.float32),
                   pltpu.SemaphoreType.DMA],
    **_MESH,
)
def _sc_deg(row2, z1, ones1, degA, degB, idxs, onesv, table, sem):
    c = lax.axis_index("c")
    s = lax.axis_index("s")
    nb = s * ROWS_PT
    pltpu.sync_copy(z1.at[pl.ds(nb, ROWS_PT)], table.at[pl.ds(nb, ROWS_PT)])
    pltpu.sync_copy(ones1, onesv)
    pltpu.sync_copy(row2.at[pl.ds((c * 16 + s) * CHUNKS_HALF, CHUNKS_HALF), :], idxs)
    plsc.subcore_barrier()

    def step(t, carry):
        for u in range(8):
            pltpu.async_copy(onesv, table.at[idxs.at[t * 8 + u]], add=True, sem=sem)
        for u in range(8):
            pltpu.make_async_copy(onesv, table.at[idxs.at[0]], sem).wait()
        return carry

    lax.fori_loop(0, CHUNKS_HALF // 8, step, 0)
    plsc.subcore_barrier()

    @pl.when(c == 0)
    def _():
        pltpu.sync_copy(table.at[pl.ds(nb, ROWS_PT)], degA.at[pl.ds(nb, ROWS_PT)])

    @pl.when(c == 1)
    def _():
        pltpu.sync_copy(table.at[pl.ds(nb, ROWS_PT)], degB.at[pl.ds(nb, ROWS_PT)])


@functools.partial(
    pl.kernel,
    out_type=jax.ShapeDtypeStruct((E_PAD,), jnp.float32),
    scratch_types=[pltpu.VMEM((CHUNKS_HALF, 128), jnp.int32),
                   pltpu.VMEM((CHUNKS_HALF * 128,), jnp.float32),
                   pltpu.SemaphoreType.DMA],
    **_MESH,
)
def _sc_disrow(row2, disf, disrow, idxs, outv, sem):
    c = lax.axis_index("c")
    s = lax.axis_index("s")
    wid = c * 16 + s
    pltpu.sync_copy(row2.at[pl.ds(wid * CHUNKS_HALF, CHUNKS_HALF), :], idxs)

    def chunk(j, carry):
        pltpu.async_copy(disf.at[idxs.at[j]], outv.at[pl.ds(j * 128, 128)], sem).wait()
        return carry

    lax.fori_loop(0, CHUNKS_HALF, chunk, 0)
    pltpu.sync_copy(outv, disrow.at[pl.ds(wid * CHUNKS_HALF * 128, CHUNKS_HALF * 128)])


@functools.partial(
    pl.kernel,
    out_type=[jax.ShapeDtypeStruct((N_PAD, 16), jnp.float32) for _ in range(4)],
    scratch_types=[pltpu.VMEM((CHUNKS_ALL // 2, 128), jnp.int32),
                   pltpu.VMEM((CHUNKS_ALL // 2, 128), jnp.int32)]
                  + [pltpu.VMEM((128, 16), jnp.float32) for _ in range(3 * _DEPTH)]
                  + [pltpu.VMEM_SHARED((N_PAD, 16), jnp.float32)]
                  + [pltpu.SemaphoreType.DMA for _ in range(2 * _DEPTH)],
    **_MESH,
)
def _sc_msgpass(hs0, hs1, hs2, hs3, eas0, eas1, eas2, eas3, row2, col2, z16,
                agg0, agg1, agg2, agg3, ri, ci, *bufs):
    DEPTH = _DEPTH
    gq = bufs[0:DEPTH]
    eq = bufs[DEPTH:2 * DEPTH]
    mq = bufs[2 * DEPTH:3 * DEPTH]
    acc = bufs[3 * DEPTH]
    sgeq = bufs[3 * DEPTH + 1:4 * DEPTH + 1]
    ssq = bufs[4 * DEPTH + 1:5 * DEPTH + 1]
    c = lax.axis_index("c")
    s = lax.axis_index("s")
    nb = s * ROWS_PT
    qc = CHUNKS_ALL // 2

    def run(cval):
        for q in range(2):
            fq = 2 * cval + q
            hs_ref = (hs0, hs1, hs2, hs3)[fq]
            eas_ref = (eas0, eas1, eas2, eas3)[fq]
            agg_ref = (agg0, agg1, agg2, agg3)[fq]
            pltpu.sync_copy(z16.at[pl.ds(nb, ROWS_PT), :], acc.at[pl.ds(nb, ROWS_PT), :])
            plsc.subcore_barrier()

            def fire(j, p, ebase0):
                pltpu.async_copy(hs_ref.at[ri.at[j]], gq[p], sgeq[p])
                pltpu.async_copy(eas_ref.at[pl.ds(ebase0 + j * 128, 128), :], eq[p], sgeq[p])

            def wait_ge(p):
                pltpu.make_async_copy(hs_ref.at[ri.at[0]], gq[p], sgeq[p]).wait()
                pltpu.make_async_copy(eas_ref.at[pl.ds(0, 128), :], eq[p], sgeq[p]).wait()

            def compute_and_scatter(j, p):
                def rowfn(r, rc):
                    for v in range(8):
                        rr = r * 8 + v
                        mq[p][rr, pl.ds(0, 16)] = jnp.maximum(
                            gq[p][rr, pl.ds(0, 16)] + eq[p][rr, pl.ds(0, 16)], 0.0)
                    return rc
                lax.fori_loop(0, 16, rowfn, 0)
                pltpu.async_copy(mq[p], acc.at[ci.at[j]], add=True, sem=ssq[p])

            def drain_sc(p):
                pltpu.make_async_copy(mq[p], acc.at[ci.at[0]], ssq[p]).wait()

            for pc in range(2):
                cbase = s * CHUNKS_ALL + pc * qc
                ebase0 = cbase * 128
                pltpu.sync_copy(row2.at[pl.ds(cbase, qc), :], ri)
                pltpu.sync_copy(col2.at[pl.ds(cbase, qc), :], ci)
                for u in range(DEPTH - 1):
                    fire(u, u, ebase0)

                def step(t, carry):
                    for u in range(DEPTH):
                        j = t * DEPTH + u
                        pl.when(t > 0)(lambda: drain_sc(u))
                        wait_ge(u)
                        jn = j + DEPTH - 1
                        pl.when(jn < qc)(lambda: fire(jn, (u + DEPTH - 1) % DEPTH, ebase0))
                        compute_and_scatter(j, u)
                    return carry

                lax.fori_loop(0, qc // DEPTH, step, 0)
                for u in range(DEPTH):
                    drain_sc(u)
            plsc.subcore_barrier()
            pltpu.sync_copy(acc.at[pl.ds(nb, ROWS_PT), :], agg_ref.at[pl.ds(nb, ROWS_PT), :])
            plsc.subcore_barrier()

    @pl.when(c == 0)
    def _():
        run(0)

    @pl.when(c == 1)
    def _():
        run(1)


def _k0_body(x, w, b, out):
    out[...] = jnp.dot(x[...], w[...].T, preferred_element_type=jnp.float32) + b[...]


def _k1_body(dA, dB, dis, inv):
    d = dA[...] + dB[...] + 1.0
    dis[...] = lax.rsqrt(d)
    inv[...] = 1.0 / d


def _k2_body(ea, dr, w, b, o0, o1, o2, o3):
    z = jnp.dot(ea[...], w[...].T, preferred_element_type=jnp.float32) + b[...]
    z = z * dr[...]
    for q, o in enumerate((o0, o1, o2, o3)):
        o[...] = z[:, q * 16:(q + 1) * 16]


def _k3_body(h, w, b, dis, hlin, hs0, hs1, hs2, hs3):
    z = jnp.dot(h[...], w[...].T, preferred_element_type=jnp.float32) + b[...]
    hlin[...] = z
    hs = z * dis[...]
    for q, o in enumerate((hs0, hs1, hs2, hs3)):
        o[...] = hs[:, q * 16:(q + 1) * 16]


def _k4a_body(hlin, agg0, agg1, agg2, agg3, hin, dis, inv, root, hnew, ps, pq):
    j = pl.program_id(0)
    agg = jnp.concatenate([agg0[...], agg1[...], agg2[...], agg3[...]],
                          axis=1) * dis[...]
    selfterm = jnp.maximum(hlin[...] + root[...], 0.0) * inv[...]
    t = hin[...] + agg + selfterm
    hnew[...] = t
    rows = j * 512 + lax.broadcasted_iota(jnp.int32, (512, 1), 0)
    tm = jnp.where(rows < N, t, 0.0)
    ps[...] = jnp.sum(tm, axis=0)[None, None, :]
    pq[...] = jnp.sum(tm * tm, axis=0)[None, None, :]


def _k4c_body(hnew, ps, pq, g, b, m, out):
    mu = jnp.sum(ps[...], axis=(0, 1)) / N
    ex2 = jnp.sum(pq[...], axis=(0, 1)) / N
    var = ex2 - mu * mu
    y = (hnew[...] - mu[None, :]) * lax.rsqrt(var[None, :] + 1e-5)
    y = y * g[...] + b[...]
    out[...] = jnp.maximum(y, y * m[...])


def _k5_body(h, bt, sums, cnt):
    j = pl.program_id(0)

    @pl.when(j == 0)
    def _():
        sums[...] = jnp.zeros_like(sums)
        cnt[...] = jnp.zeros_like(cnt)

    oh = (bt[...] == lax.broadcasted_iota(jnp.int32, (1, G), 1)).astype(jnp.float32)
    dn = (((0,), (0,)), ((), ()))
    sums[...] += lax.dot_general(oh, h[...], dn, preferred_element_type=jnp.float32)
    cnt[...] += lax.dot_general(oh, jnp.ones((512, EMB), jnp.float32), dn,
                                preferred_element_type=jnp.float32)


def _k6_body(sums, cnt, w1, b1, w2, b2, w3, b3, out):
    hg = sums[...] * (1.0 / jnp.maximum(cnt[...], 1.0))
    dn = (((1,), (1,)), ((), ()))
    z1 = jnp.maximum(lax.dot_general(hg, w1[...], dn, preferred_element_type=jnp.float32) + b1[...], 0.0)
    z2 = jnp.maximum(lax.dot_general(z1, w2[...], dn, preferred_element_type=jnp.float32) + b2[...], 0.0)
    out[...] = lax.dot_general(z2, w3[...], dn, preferred_element_type=jnp.float32) + b3[0, 0]


def _k6_specs():
    H = EMB // 2
    return dict(
        in_specs=[_full((G, EMB)), _full((G, EMB)), _full((H, EMB)), _full((1, H)),
                  _full((H, H)), _full((1, H)), _full((G, H)), _full((1, 1))],
        out_specs=_full((G, G)),
        out_shape=jax.ShapeDtypeStruct((G, G), _f32),
    )


def _full(shape):
    return pl.BlockSpec(shape, lambda *a: tuple(0 for _ in shape))


def _rows(shape):
    nd = len(shape)
    return pl.BlockSpec(shape, lambda j: (j,) + tuple(0 for _ in range(nd - 1)))


_f32 = jnp.float32


def _tc_embed(x, w, b):
    return pl.pallas_call(
        _k0_body, grid=(NB,),
        in_specs=[_rows((512, 40)), _full((EMB, 40)), _full((1, EMB))],
        out_specs=_rows((512, EMB)),
        out_shape=jax.ShapeDtypeStruct((N_PAD, EMB), _f32),
    )(x, w, b)


def _tc_dis(dA, dB):
    return pl.pallas_call(
        _k1_body, grid=(NB,),
        in_specs=[_rows((512, 1)), _rows((512, 1))],
        out_specs=[_rows((512, 1)), _rows((512, 1))],
        out_shape=[jax.ShapeDtypeStruct((N_PAD, 1), _f32),
                   jax.ShapeDtypeStruct((N_PAD, 1), _f32)],
    )(dA, dB)


def _tc_eas(ea, dr, w, b):
    return pl.pallas_call(
        _k2_body, grid=(EB,),
        in_specs=[_rows((1024, 10)), _rows((1024, 1)), _full((EMB, 10)), _full((1, EMB))],
        out_specs=[_rows((1024, 16)) for _ in range(4)],
        out_shape=[jax.ShapeDtypeStruct((E_PAD, 16), _f32) for _ in range(4)],
    )(ea, dr, w, b)


def _tc_lin(h, w, b, dis):
    return pl.pallas_call(
        _k3_body, grid=(NB,),
        in_specs=[_rows((512, EMB)), _full((EMB, EMB)), _full((1, EMB)), _rows((512, 1))],
        out_specs=[_rows((512, EMB))] + [_rows((512, 16)) for _ in range(4)],
        out_shape=[jax.ShapeDtypeStruct((N_PAD, EMB), _f32)]
                  + [jax.ShapeDtypeStruct((N_PAD, 16), _f32) for _ in range(4)],
    )(h, w, b, dis)


def _tc_combine(hlin, aggq, hin, dis, inv, root):
    return pl.pallas_call(
        _k4a_body, grid=(NB,),
        in_specs=[_rows((512, EMB))] + [_rows((512, 16)) for _ in range(4)]
                 + [_rows((512, EMB)), _rows((512, 1)), _rows((512, 1)), _full((1, EMB))],
        out_specs=[_rows((512, EMB)),
                   pl.BlockSpec((1, 1, EMB), lambda j: (j, 0, 0)),
                   pl.BlockSpec((1, 1, EMB), lambda j: (j, 0, 0))],
        out_shape=[jax.ShapeDtypeStruct((N_PAD, EMB), _f32),
                   jax.ShapeDtypeStruct((NB, 1, EMB), _f32),
                   jax.ShapeDtypeStruct((NB, 1, EMB), _f32)],
    )(hlin, *aggq, hin, dis, inv, root)


def _tc_bn(hnew, ps, pq, g, b, m):
    return pl.pallas_call(
        _k4c_body, grid=(NB,),
        in_specs=[_rows((512, EMB)), _full((NB, 1, EMB)), _full((NB, 1, EMB)),
                  _full((1, EMB)), _full((1, EMB)), _full((1, EMB))],
        out_specs=_rows((512, EMB)),
        out_shape=jax.ShapeDtypeStruct((N_PAD, EMB), _f32),
    )(hnew, ps, pq, g, b, m)


def _tc_pool(h, bt):
    return pl.pallas_call(
        _k5_body, grid=(NB,),
        in_specs=[_rows((512, EMB)), _rows((512, 1))],
        out_specs=[_full((G, EMB)), _full((G, EMB))],
        out_shape=[jax.ShapeDtypeStruct((G, EMB), _f32),
                   jax.ShapeDtypeStruct((G, EMB), _f32)],
    )(h, bt)


def _tc_mlp(sums, cnt, w1, b1, w2, b2, w3, b3):
    return pl.pallas_call(_k6_body, grid=(1,), **_k6_specs())(
        sums, cnt, w1, b1, w2, b2, w3, b3)


@jax.jit
def _stage_pre(x_p, row2, ea_p, params):
    z1 = jnp.zeros((N_PAD,), _f32)
    ones1 = jnp.ones((128,), _f32)
    degA, degB = _sc_deg(row2, z1, ones1)
    dis2, inv2 = _tc_dis(degA.reshape(N_PAD, 1), degB.reshape(N_PAD, 1))
    disrow = _sc_disrow(row2, dis2.reshape(N_PAD))
    easq = _tc_eas(ea_p, disrow.reshape(E_PAD, 1),
                   params['edge_emb_W'], params['edge_emb_b'].reshape(1, EMB))
    h = _tc_embed(x_p, params['x_emb_W'], params['x_emb_b'].reshape(1, EMB))
    return h, dis2, inv2, easq


@jax.jit
def _stage_layers(h, dis2, inv2, easq, row2, col2, bt_p, params):
    z16 = jnp.zeros((N_PAD, 16), _f32)
    lps = params['layers']
    Ws = jnp.stack([lp['lin_W'] for lp in lps])
    bs = jnp.stack([lp['lin_b'].reshape(1, EMB) for lp in lps])
    roots = jnp.stack([lp['root'] for lp in lps])
    gs = jnp.stack([lp['bn_g'].reshape(1, EMB) for lp in lps])
    betas = jnp.stack([lp['bn_b'].reshape(1, EMB) for lp in lps])
    relu_m = jnp.concatenate([jnp.zeros((L - 1, 1, EMB), _f32),
                              jnp.ones((1, 1, EMB), _f32)])

    for l in range(L):
        idx = lambda a: a[l]
        hlin, *hsq = _tc_lin(h, idx(Ws), idx(bs), dis2)
        aggq = _sc_msgpass(*hsq, *easq, row2, col2, z16)
        hnew, ps, pq = _tc_combine(hlin, aggq, h, dis2, inv2, idx(roots))
        h = _tc_bn(hnew, ps, pq, idx(gs), idx(betas), idx(relu_m))

    sums, cnt = _tc_pool(h, bt_p)
    (w1, b1), (w2, b2), (w3, b3) = params['pred']
    H = EMB // 2
    w3p = jnp.pad(w3, ((0, G - 1), (0, 0)))
    full = _tc_mlp(sums, cnt, w1, b1.reshape(1, H), w2, b2.reshape(1, H),
                   w3p, b3.reshape(1, 1))
    return full[:, :1]


def kernel(x, edge_index, edge_attr, batch, params):
    x_p = jnp.pad(x, ((0, N_PAD - N), (0, 0)))
    row = jnp.pad(edge_index[0], (0, E_PAD - E), constant_values=JUNK)
    col = jnp.pad(edge_index[1], (0, E_PAD - E), constant_values=JUNK)
    ea_p = jnp.pad(edge_attr, ((0, E_PAD - E), (0, 0)))
    bt_p = jnp.pad(batch, (0, N_PAD - N), constant_values=G).reshape(N_PAD, 1)
    row2 = row.reshape(E_PAD // 128, 128)
    col2 = col.reshape(E_PAD // 128, 128)
    h, dis2, inv2, easq = _stage_pre(x_p, row2, ea_p, params)
    return _stage_layers(h, dis2, inv2, easq, row2, col2, bt_p, params)
